# baseline (speedup 1.0000x reference)
import functools

import jax
import jax.numpy as jnp
from jax import lax
from jax.experimental import pallas as pl
from jax.experimental.pallas import tpu as pltpu
from jax.experimental.pallas import tpu_sc as plsc

C = 100000
F = 128
B = 1024
BETA = 0.05
K = 6
BC = 2048
NB = 49
CP = NB * BC
CF = (NB - 1) * BC
NCH = BC // 128
HF = F // 2
RT = 64
NEG = -3.0e38
PADB = -1.0e30


def _gather_target_rows(em, targets):
    info = plsc.get_sparse_core_info()
    nw = info.num_cores * info.num_subcores
    bpw = B // nw
    mesh = plsc.VectorSubcoreMesh(core_axis_name="c", subcore_axis_name="s")

    @functools.partial(
        pl.kernel,
        mesh=mesh,
        out_type=jax.ShapeDtypeStruct((B, F), jnp.float32),
        scratch_types=[
            pltpu.VMEM((bpw,), jnp.int32),
            pltpu.VMEM((bpw, F), jnp.float32),
            pltpu.SemaphoreType.DMA,
        ],
    )
    def gather_kernel(em_hbm, idx_hbm, out_hbm, idx_v, rows_v, sem):
        wid = lax.axis_index("s") * info.num_cores + lax.axis_index("c")
        base = wid * bpw
        pltpu.sync_copy(idx_hbm.at[pl.ds(base, bpw)], idx_v)
        pltpu.async_copy(em_hbm.at[idx_v], rows_v, sem).wait()
        pltpu.sync_copy(rows_v, out_hbm.at[pl.ds(base, bpw)])

    return gather_kernel(em, targets)


def _body(xs_ref, em_ref, tail_ref, bias_ref, emt_ref, out_ref,
          m_ref, acc_ref, kb0, kb1, tau0, tau1, ks0, ks1, ps0, ps1):
    p = pl.program_id(0)
    j = pl.program_id(1)
    xs = xs_ref[...]
    x0 = xs[:, :HF]
    x1 = xs[:, HF:]
    dn = (((1,), (1,)), ((), ()))

    def sweep0(s0, s1):
        sim = s0 + s1
        bmax = jnp.max(sim, axis=1, keepdims=True)
        m_old = m_ref[...]
        m_new = jnp.maximum(m_old, bmax)
        ex = jnp.exp(sim - m_new)
        acc_ref[...] = (acc_ref[...] * jnp.exp(m_old - m_new)
                        + jnp.sum(ex, axis=1, keepdims=True))
        m_ref[...] = m_new
        for rt in range(B // RT):
            lo = rt * RT
            hi_r = lo + RT
            for s, kb in ((s0, kb0), (s1, kb1)):
                r = [kb[i, lo:hi_r, :] for i in range(K)]
                for c in range(0, NCH, 2):
                    ea = s[lo:hi_r, c * 128:(c + 1) * 128]
                    eb = s[lo:hi_r, (c + 1) * 128:(c + 2) * 128]
                    h = jnp.maximum(ea, eb)
                    l = jnp.minimum(ea, eb)
                    e0 = jnp.maximum(r[0], h)
                    m = jnp.minimum(r[0], h)
                    e1 = jnp.maximum(r[2], m)
                    m = jnp.minimum(r[2], m)
                    e2 = jnp.maximum(r[4], m)
                    e3 = jnp.minimum(r[4], m)
                    o0 = jnp.maximum(r[1], l)
                    n = jnp.minimum(r[1], l)
                    o1 = jnp.maximum(r[3], n)
                    n = jnp.minimum(r[3], n)
                    o2 = jnp.maximum(r[5], n)
                    z1 = jnp.maximum(o0, e1)
                    z2 = jnp.minimum(o0, e1)
                    z3 = jnp.maximum(o1, e2)
                    z4 = jnp.minimum(o1, e2)
                    z5 = jnp.maximum(o2, e3)
                    r = [e0, z1, z2, z3, z4, z5]
                for i in range(K):
                    kb[i, lo:hi_r, :] = r[i]

    def sweep1(s0, s1):
        t0 = tau0[...]
        t1 = tau1[...]
        ps0[...] = ps0[...] + jnp.sum(
            jnp.where(s0 >= t0, s1, 0.0), axis=1, keepdims=True)
        ps1[...] = ps1[...] + jnp.sum(
            jnp.where(s1 >= t1, s0, 0.0), axis=1, keepdims=True)

    @pl.when(j < NB - 1)
    def _clean():
        emb = em_ref[...]
        s0 = lax.dot_general(x0, emb[:, :HF], dn,
                             preferred_element_type=jnp.float32)
        s1 = lax.dot_general(x1, emb[:, HF:], dn,
                             preferred_element_type=jnp.float32)

        @pl.when(jnp.logical_and(p == 0, j == 0))
        def _init():
            m_ref[...] = jnp.full((B, 1), NEG, jnp.float32)
            acc_ref[...] = jnp.zeros((B, 1), jnp.float32)
            kb0[...] = jnp.full((K, B, 128), NEG, jnp.float32)
            kb1[...] = jnp.full((K, B, 128), NEG, jnp.float32)
            ps0[...] = jnp.zeros((B, 1), jnp.float32)
            ps1[...] = jnp.zeros((B, 1), jnp.float32)

        @pl.when(p == 0)
        def _p0():
            sweep0(s0, s1)

        @pl.when(p == 1)
        def _p1():
            @pl.when(j == 0)
            def _finalize_tau():
                for kb, tau, ks in ((kb0, tau0, ks0), (kb1, tau1, ks1)):
                    cand = jnp.concatenate(
                        [kb[i, :, :] for i in range(K)], axis=1)
                    ksum = jnp.zeros((B, 1), jnp.float32)
                    for t in range(K):
                        mt = jnp.max(cand, axis=1, keepdims=True)
                        ksum = ksum + mt
                        if t < K - 1:
                            cand = jnp.where(cand == mt, NEG, cand)
                        else:
                            tau[...] = mt
                    ks[...] = ksum

            sweep1(s0, s1)

    @pl.when(j == NB - 1)
    def _tail():
        emb = tail_ref[...]
        bias = bias_ref[...]
        s0 = lax.dot_general(x0, emb[:, :HF], dn,
                             preferred_element_type=jnp.float32) + bias
        s1 = lax.dot_general(x1, emb[:, HF:], dn,
                             preferred_element_type=jnp.float32) + bias

        @pl.when(p == 0)
        def _p0():
            sweep0(s0, s1)

        @pl.when(p == 1)
        def _p1():
            sweep1(s0, s1)
            lse = m_ref[...] + jnp.log(acc_ref[...])
            prod = xs * emt_ref[...]
            st0 = jnp.sum(prod[:, :HF], axis=1, keepdims=True)
            st1 = jnp.sum(prod[:, HF:], axis=1, keepdims=True)
            sim_t = st0 + st1
            logp_t = sim_t - lse
            sum_logp0 = ks0[...] + ps0[...] - K * lse
            sum_logp1 = ks1[...] + ps1[...] - K * lse
            in0 = (st0 >= tau0[...]).astype(jnp.float32)
            in1 = (st1 >= tau1[...]).astype(jnp.float32)
            inv_k = 1.0 / K
            l0 = -logp_t - inv_k * (sum_logp0 - logp_t * in0)
            l1 = -logp_t - inv_k * (sum_logp1 - logp_t * in1)
            smooth = jnp.sum(0.5 * (l0 + l1)) * (1.0 / B)
            plain = jnp.sum(-logp_t) * (1.0 / B)
            lane = lax.broadcasted_iota(jnp.int32, (1, 128), 1)
            out_ref[...] = jnp.where(
                lane == 0, smooth, jnp.where(lane == 1, plain, 0.0))


def _tc_losses(xs, em, tail, bias, emt, interpret=False):
    out = pl.pallas_call(
        _body,
        grid=(2, NB),
        in_specs=[
            pl.BlockSpec((B, F), lambda p, j: (0, 0)),
            pl.BlockSpec((BC, F), lambda p, j: (jnp.minimum(j, NB - 2), 0)),
            pl.BlockSpec((BC, F), lambda p, j: (0, 0)),
            pl.BlockSpec((1, BC), lambda p, j: (0, 0)),
            pl.BlockSpec((B, F), lambda p, j: (0, 0)),
        ],
        out_specs=pl.BlockSpec((1, 128), lambda p, j: (0, 0)),
        out_shape=jax.ShapeDtypeStruct((1, 128), jnp.float32),
        scratch_shapes=[
            pltpu.VMEM((B, 1), jnp.float32),
            pltpu.VMEM((B, 1), jnp.float32),
            pltpu.VMEM((K, B, 128), jnp.float32),
            pltpu.VMEM((K, B, 128), jnp.float32),
            pltpu.VMEM((B, 1), jnp.float32),
            pltpu.VMEM((B, 1), jnp.float32),
            pltpu.VMEM((B, 1), jnp.float32),
            pltpu.VMEM((B, 1), jnp.float32),
            pltpu.VMEM((B, 1), jnp.float32),
            pltpu.VMEM((B, 1), jnp.float32),
        ],
        compiler_params=pltpu.CompilerParams(
            dimension_semantics=("arbitrary", "arbitrary")),
        interpret=interpret,
    )(xs, em, tail, bias, emt)
    return out[0, 0], out[0, 1]


def _prep(inputs, em):
    xs = inputs * (1.0 / BETA)
    tail = jnp.pad(em[CF:], ((0, CP - C), (0, 0)))
    col = jnp.arange(BC, dtype=jnp.int32)[None, :]
    bias = jnp.where(col < C - CF, 0.0, PADB).astype(jnp.float32)
    return xs, tail, bias


def kernel(inputs, targets, em, epoch):
    xs, tail, bias = _prep(inputs, em)
    emt = _gather_target_rows(em, targets)
    smooth, plain = _tc_losses(xs, em, tail, bias, emt)
    return jnp.where(epoch > 4, smooth, plain)


# speedup vs baseline: 1.0057x; 1.0057x over previous
import functools

import jax
import jax.numpy as jnp
from jax import lax
from jax.experimental import pallas as pl
from jax.experimental.pallas import tpu as pltpu
from jax.experimental.pallas import tpu_sc as plsc

C = 100000
F = 128
B = 1024
BETA = 0.05
K = 6
BC = 2048
NB = 49
CP = NB * BC
CF = (NB - 1) * BC
NCH = BC // 128
HF = F // 2
RT = 64
NEG = -3.0e38
PADB = -1.0e30
LOG2E = 1.4426950408889634
LN2 = 0.6931471805599453


def _gather_target_rows(em, targets):
    info = plsc.get_sparse_core_info()
    nw = info.num_cores * info.num_subcores
    bpw = B // nw
    mesh = plsc.VectorSubcoreMesh(core_axis_name="c", subcore_axis_name="s")

    @functools.partial(
        pl.kernel,
        mesh=mesh,
        out_type=jax.ShapeDtypeStruct((B, F), jnp.float32),
        scratch_types=[
            pltpu.VMEM((bpw,), jnp.int32),
            pltpu.VMEM((bpw, F), jnp.float32),
            pltpu.SemaphoreType.DMA,
        ],
    )
    def gather_kernel(em_hbm, idx_hbm, out_hbm, idx_v, rows_v, sem):
        wid = lax.axis_index("s") * info.num_cores + lax.axis_index("c")
        base = wid * bpw
        pltpu.sync_copy(idx_hbm.at[pl.ds(base, bpw)], idx_v)
        pltpu.async_copy(em_hbm.at[idx_v], rows_v, sem).wait()
        pltpu.sync_copy(rows_v, out_hbm.at[pl.ds(base, bpw)])

    return gather_kernel(em, targets)


def _body(xs_ref, em_ref, tail_ref, bias_ref, emt_ref, out_ref,
          m_ref, acc_ref, kb0, kb1, tau0, tau1, ks0, ks1, ps0, ps1):
    p = pl.program_id(0)
    j = pl.program_id(1)
    xs = xs_ref[...]
    x0 = xs[:, :HF]
    x1 = xs[:, HF:]
    dn = (((1,), (1,)), ((), ()))

    def sweep0(s0, s1):
        sim = s0 + s1
        bmax = jnp.max(sim, axis=1, keepdims=True)
        m_old = m_ref[...]
        m_new = jnp.maximum(m_old, bmax)
        ex = jnp.exp2(sim - m_new)
        ones = jnp.ones((BC,), jnp.float32)
        dnv = (((1,), (0,)), ((), ()))
        exs = lax.dot_general(ex, ones, dnv,
                              preferred_element_type=jnp.float32)
        acc_ref[...] = (acc_ref[...] * jnp.exp2(m_old - m_new)
                        + exs[:, None])
        m_ref[...] = m_new
        for rt in range(B // RT):
            lo = rt * RT
            hi_r = lo + RT
            for s, kb in ((s0, kb0), (s1, kb1)):
                r = [kb[i, lo:hi_r, :] for i in range(K)]
                for c in range(0, NCH, 2):
                    ea = s[lo:hi_r, c * 128:(c + 1) * 128]
                    eb = s[lo:hi_r, (c + 1) * 128:(c + 2) * 128]
                    h = jnp.maximum(ea, eb)
                    l = jnp.minimum(ea, eb)
                    e0 = jnp.maximum(r[0], h)
                    m = jnp.minimum(r[0], h)
                    e1 = jnp.maximum(r[2], m)
                    m = jnp.minimum(r[2], m)
                    e2 = jnp.maximum(r[4], m)
                    e3 = jnp.minimum(r[4], m)
                    o0 = jnp.maximum(r[1], l)
                    n = jnp.minimum(r[1], l)
                    o1 = jnp.maximum(r[3], n)
                    n = jnp.minimum(r[3], n)
                    o2 = jnp.maximum(r[5], n)
                    z1 = jnp.maximum(o0, e1)
                    z2 = jnp.minimum(o0, e1)
                    z3 = jnp.maximum(o1, e2)
                    z4 = jnp.minimum(o1, e2)
                    z5 = jnp.maximum(o2, e3)
                    r = [e0, z1, z2, z3, z4, z5]
                for i in range(K):
                    kb[i, lo:hi_r, :] = r[i]

    def sweep1(s0, s1):
        t0 = tau0[...]
        t1 = tau1[...]
        ones = jnp.ones((BC,), jnp.float32)
        dnv = (((1,), (0,)), ((), ()))
        q0 = lax.dot_general(jnp.where(s0 >= t0, s1, 0.0), ones, dnv,
                             preferred_element_type=jnp.float32)
        q1 = lax.dot_general(jnp.where(s1 >= t1, s0, 0.0), ones, dnv,
                             preferred_element_type=jnp.float32)
        ps0[...] = ps0[...] + q0[:, None]
        ps1[...] = ps1[...] + q1[:, None]

    @pl.when(j < NB - 1)
    def _clean():
        emb = em_ref[...]
        s0 = lax.dot_general(x0, emb[:, :HF], dn,
                             preferred_element_type=jnp.float32)
        s1 = lax.dot_general(x1, emb[:, HF:], dn,
                             preferred_element_type=jnp.float32)

        @pl.when(jnp.logical_and(p == 0, j == 0))
        def _init():
            m_ref[...] = jnp.full((B, 1), NEG, jnp.float32)
            acc_ref[...] = jnp.zeros((B, 1), jnp.float32)
            kb0[...] = jnp.full((K, B, 128), NEG, jnp.float32)
            kb1[...] = jnp.full((K, B, 128), NEG, jnp.float32)
            ps0[...] = jnp.zeros((B, 1), jnp.float32)
            ps1[...] = jnp.zeros((B, 1), jnp.float32)

        @pl.when(p == 0)
        def _p0():
            sweep0(s0, s1)

        @pl.when(p == 1)
        def _p1():
            @pl.when(j == 0)
            def _finalize_tau():
                for kb, tau, ks in ((kb0, tau0, ks0), (kb1, tau1, ks1)):
                    cand = jnp.concatenate(
                        [kb[i, :, :] for i in range(K)], axis=1)
                    ksum = jnp.zeros((B, 1), jnp.float32)
                    for t in range(K):
                        mt = jnp.max(cand, axis=1, keepdims=True)
                        ksum = ksum + mt
                        if t < K - 1:
                            cand = jnp.where(cand == mt, NEG, cand)
                        else:
                            tau[...] = mt
                    ks[...] = ksum

            sweep1(s0, s1)

    @pl.when(j == NB - 1)
    def _tail():
        emb = tail_ref[...]
        bias = bias_ref[...]
        s0 = lax.dot_general(x0, emb[:, :HF], dn,
                             preferred_element_type=jnp.float32) + bias
        s1 = lax.dot_general(x1, emb[:, HF:], dn,
                             preferred_element_type=jnp.float32) + bias

        @pl.when(p == 0)
        def _p0():
            sweep0(s0, s1)

        @pl.when(p == 1)
        def _p1():
            sweep1(s0, s1)
            lse = m_ref[...] + jnp.log(acc_ref[...]) * LOG2E
            prod = xs * emt_ref[...]
            st0 = jnp.sum(prod[:, :HF], axis=1, keepdims=True)
            st1 = jnp.sum(prod[:, HF:], axis=1, keepdims=True)
            sim_t = st0 + st1
            logp_t = sim_t - lse
            sum_logp0 = ks0[...] + ps0[...] - K * lse
            sum_logp1 = ks1[...] + ps1[...] - K * lse
            in0 = (st0 >= tau0[...]).astype(jnp.float32)
            in1 = (st1 >= tau1[...]).astype(jnp.float32)
            inv_k = 1.0 / K
            l0 = -logp_t - inv_k * (sum_logp0 - logp_t * in0)
            l1 = -logp_t - inv_k * (sum_logp1 - logp_t * in1)
            smooth = jnp.sum(0.5 * (l0 + l1)) * (LN2 / B)
            plain = jnp.sum(-logp_t) * (LN2 / B)
            lane = lax.broadcasted_iota(jnp.int32, (1, 128), 1)
            out_ref[...] = jnp.where(
                lane == 0, smooth, jnp.where(lane == 1, plain, 0.0))


def _tc_losses(xs, em, tail, bias, emt, interpret=False):
    out = pl.pallas_call(
        _body,
        grid=(2, NB),
        in_specs=[
            pl.BlockSpec((B, F), lambda p, j: (0, 0)),
            pl.BlockSpec((BC, F), lambda p, j: (jnp.minimum(j, NB - 2), 0)),
            pl.BlockSpec((BC, F), lambda p, j: (0, 0)),
            pl.BlockSpec((1, BC), lambda p, j: (0, 0)),
            pl.BlockSpec((B, F), lambda p, j: (0, 0)),
        ],
        out_specs=pl.BlockSpec((1, 128), lambda p, j: (0, 0)),
        out_shape=jax.ShapeDtypeStruct((1, 128), jnp.float32),
        scratch_shapes=[
            pltpu.VMEM((B, 1), jnp.float32),
            pltpu.VMEM((B, 1), jnp.float32),
            pltpu.VMEM((K, B, 128), jnp.float32),
            pltpu.VMEM((K, B, 128), jnp.float32),
            pltpu.VMEM((B, 1), jnp.float32),
            pltpu.VMEM((B, 1), jnp.float32),
            pltpu.VMEM((B, 1), jnp.float32),
            pltpu.VMEM((B, 1), jnp.float32),
            pltpu.VMEM((B, 1), jnp.float32),
            pltpu.VMEM((B, 1), jnp.float32),
        ],
        compiler_params=pltpu.CompilerParams(
            dimension_semantics=("arbitrary", "arbitrary")),
        interpret=interpret,
    )(xs, em, tail, bias, emt)
    return out[0, 0], out[0, 1]


def _prep(inputs, em):
    xs = inputs * (LOG2E / BETA)
    tail = jnp.pad(em[CF:], ((0, CP - C), (0, 0)))
    col = jnp.arange(BC, dtype=jnp.int32)[None, :]
    bias = jnp.where(col < C - CF, 0.0, PADB).astype(jnp.float32)
    return xs, tail, bias


def kernel(inputs, targets, em, epoch):
    xs, tail, bias = _prep(inputs, em)
    emt = _gather_target_rows(em, targets)
    smooth, plain = _tc_losses(xs, em, tail, bias, emt)
    return jnp.where(epoch > 4, smooth, plain)


# speedup vs baseline: 1.0333x; 1.0274x over previous
import functools

import jax
import jax.numpy as jnp
from jax import lax
from jax.experimental import pallas as pl
from jax.experimental.pallas import tpu as pltpu
from jax.experimental.pallas import tpu_sc as plsc

C = 100000
F = 128
B = 1024
BETA = 0.05
K = 6
BC = 2048
NB = 49
CP = NB * BC
CF = (NB - 1) * BC
NCH = BC // 128
HF = F // 2
RT = 32
NEG = -3.0e38
PADB = -1.0e30
LOG2E = 1.4426950408889634
LN2 = 0.6931471805599453


def _gather_target_rows(em, targets):
    info = plsc.get_sparse_core_info()
    nw = info.num_cores * info.num_subcores
    bpw = B // nw
    mesh = plsc.VectorSubcoreMesh(core_axis_name="c", subcore_axis_name="s")

    @functools.partial(
        pl.kernel,
        mesh=mesh,
        out_type=jax.ShapeDtypeStruct((B, F), jnp.float32),
        scratch_types=[
            pltpu.VMEM((bpw,), jnp.int32),
            pltpu.VMEM((bpw, F), jnp.float32),
            pltpu.SemaphoreType.DMA,
        ],
    )
    def gather_kernel(em_hbm, idx_hbm, out_hbm, idx_v, rows_v, sem):
        wid = lax.axis_index("s") * info.num_cores + lax.axis_index("c")
        base = wid * bpw
        pltpu.sync_copy(idx_hbm.at[pl.ds(base, bpw)], idx_v)
        pltpu.async_copy(em_hbm.at[idx_v], rows_v, sem).wait()
        pltpu.sync_copy(rows_v, out_hbm.at[pl.ds(base, bpw)])

    return gather_kernel(em, targets)


def _body(xs_ref, em_ref, tail_ref, bias_ref, emt_ref, out_ref,
          m_ref, acc_ref, kb0, kb1, tau0, tau1, ks0, ks1, ps0, ps1):
    p = pl.program_id(0)
    j = pl.program_id(1)
    xs = xs_ref[...]
    x0 = xs[:, :HF]
    x1 = xs[:, HF:]
    dn = (((1,), (1,)), ((), ()))

    def sweep0(s0, s1):
        sim = s0 + s1
        bmax = jnp.max(sim, axis=1, keepdims=True)
        m_old = m_ref[...]
        m_new = jnp.maximum(m_old, bmax)
        ex = jnp.exp2(sim - m_new)
        ones = jnp.ones((BC,), jnp.float32)
        dnv = (((1,), (0,)), ((), ()))
        exs = lax.dot_general(ex, ones, dnv,
                              preferred_element_type=jnp.float32)
        acc_ref[...] = (acc_ref[...] * jnp.exp2(m_old - m_new)
                        + exs[:, None])
        m_ref[...] = m_new
        for rt in range(B // RT):
            lo = rt * RT
            hi_r = lo + RT
            for s, kb in ((s0, kb0), (s1, kb1)):
                r = [kb[i, lo:hi_r, :] for i in range(K)]
                for c in range(0, NCH, 2):
                    ea = s[lo:hi_r, c * 128:(c + 1) * 128]
                    eb = s[lo:hi_r, (c + 1) * 128:(c + 2) * 128]
                    h = jnp.maximum(ea, eb)
                    l = jnp.minimum(ea, eb)
                    e0 = jnp.maximum(r[0], h)
                    m = jnp.minimum(r[0], h)
                    e1 = jnp.maximum(r[2], m)
                    m = jnp.minimum(r[2], m)
                    e2 = jnp.maximum(r[4], m)
                    e3 = jnp.minimum(r[4], m)
                    o0 = jnp.maximum(r[1], l)
                    n = jnp.minimum(r[1], l)
                    o1 = jnp.maximum(r[3], n)
                    n = jnp.minimum(r[3], n)
                    o2 = jnp.maximum(r[5], n)
                    z1 = jnp.maximum(o0, e1)
                    z2 = jnp.minimum(o0, e1)
                    z3 = jnp.maximum(o1, e2)
                    z4 = jnp.minimum(o1, e2)
                    z5 = jnp.maximum(o2, e3)
                    r = [e0, z1, z2, z3, z4, z5]
                for i in range(K):
                    kb[i, lo:hi_r, :] = r[i]

    def sweep1(s0, s1):
        t0 = tau0[...]
        t1 = tau1[...]
        ones = jnp.ones((BC,), jnp.float32)
        dnv = (((1,), (0,)), ((), ()))
        q0 = lax.dot_general(jnp.where(s0 >= t0, s1, 0.0), ones, dnv,
                             preferred_element_type=jnp.float32)
        q1 = lax.dot_general(jnp.where(s1 >= t1, s0, 0.0), ones, dnv,
                             preferred_element_type=jnp.float32)
        ps0[...] = ps0[...] + q0[:, None]
        ps1[...] = ps1[...] + q1[:, None]

    @pl.when(j < NB - 1)
    def _clean():
        emb = em_ref[...]
        s0 = lax.dot_general(x0, emb[:, :HF], dn,
                             preferred_element_type=jnp.float32)
        s1 = lax.dot_general(x1, emb[:, HF:], dn,
                             preferred_element_type=jnp.float32)

        @pl.when(jnp.logical_and(p == 0, j == 0))
        def _init():
            m_ref[...] = jnp.full((B, 1), NEG, jnp.float32)
            acc_ref[...] = jnp.zeros((B, 1), jnp.float32)
            kb0[...] = jnp.full((K, B, 128), NEG, jnp.float32)
            kb1[...] = jnp.full((K, B, 128), NEG, jnp.float32)
            ps0[...] = jnp.zeros((B, 1), jnp.float32)
            ps1[...] = jnp.zeros((B, 1), jnp.float32)

        @pl.when(p == 0)
        def _p0():
            sweep0(s0, s1)

        @pl.when(p == 1)
        def _p1():
            @pl.when(j == 0)
            def _finalize_tau():
                for kb, tau, ks in ((kb0, tau0, ks0), (kb1, tau1, ks1)):
                    cand = jnp.concatenate(
                        [kb[i, :, :] for i in range(K)], axis=1)
                    ksum = jnp.zeros((B, 1), jnp.float32)
                    for t in range(K):
                        mt = jnp.max(cand, axis=1, keepdims=True)
                        ksum = ksum + mt
                        if t < K - 1:
                            cand = jnp.where(cand == mt, NEG, cand)
                        else:
                            tau[...] = mt
                    ks[...] = ksum

            sweep1(s0, s1)

    @pl.when(j == NB - 1)
    def _tail():
        emb = tail_ref[...]
        bias = bias_ref[...]
        s0 = lax.dot_general(x0, emb[:, :HF], dn,
                             preferred_element_type=jnp.float32) + bias
        s1 = lax.dot_general(x1, emb[:, HF:], dn,
                             preferred_element_type=jnp.float32) + bias

        @pl.when(p == 0)
        def _p0():
            sweep0(s0, s1)

        @pl.when(p == 1)
        def _p1():
            sweep1(s0, s1)
            lse = m_ref[...] + jnp.log(acc_ref[...]) * LOG2E
            prod = xs * emt_ref[...]
            st0 = jnp.sum(prod[:, :HF], axis=1, keepdims=True)
            st1 = jnp.sum(prod[:, HF:], axis=1, keepdims=True)
            sim_t = st0 + st1
            logp_t = sim_t - lse
            sum_logp0 = ks0[...] + ps0[...] - K * lse
            sum_logp1 = ks1[...] + ps1[...] - K * lse
            in0 = (st0 >= tau0[...]).astype(jnp.float32)
            in1 = (st1 >= tau1[...]).astype(jnp.float32)
            inv_k = 1.0 / K
            l0 = -logp_t - inv_k * (sum_logp0 - logp_t * in0)
            l1 = -logp_t - inv_k * (sum_logp1 - logp_t * in1)
            smooth = jnp.sum(0.5 * (l0 + l1)) * (LN2 / B)
            plain = jnp.sum(-logp_t) * (LN2 / B)
            lane = lax.broadcasted_iota(jnp.int32, (1, 128), 1)
            out_ref[...] = jnp.where(
                lane == 0, smooth, jnp.where(lane == 1, plain, 0.0))


def _tc_losses(xs, em, tail, bias, emt, interpret=False):
    out = pl.pallas_call(
        _body,
        grid=(2, NB),
        in_specs=[
            pl.BlockSpec((B, F), lambda p, j: (0, 0)),
            pl.BlockSpec((BC, F), lambda p, j: (jnp.minimum(j, NB - 2), 0)),
            pl.BlockSpec((BC, F), lambda p, j: (0, 0)),
            pl.BlockSpec((1, BC), lambda p, j: (0, 0)),
            pl.BlockSpec((B, F), lambda p, j: (0, 0)),
        ],
        out_specs=pl.BlockSpec((1, 128), lambda p, j: (0, 0)),
        out_shape=jax.ShapeDtypeStruct((1, 128), jnp.float32),
        scratch_shapes=[
            pltpu.VMEM((B, 1), jnp.float32),
            pltpu.VMEM((B, 1), jnp.float32),
            pltpu.VMEM((K, B, 128), jnp.float32),
            pltpu.VMEM((K, B, 128), jnp.float32),
            pltpu.VMEM((B, 1), jnp.float32),
            pltpu.VMEM((B, 1), jnp.float32),
            pltpu.VMEM((B, 1), jnp.float32),
            pltpu.VMEM((B, 1), jnp.float32),
            pltpu.VMEM((B, 1), jnp.float32),
            pltpu.VMEM((B, 1), jnp.float32),
        ],
        compiler_params=pltpu.CompilerParams(
            dimension_semantics=("arbitrary", "arbitrary")),
        interpret=interpret,
    )(xs, em, tail, bias, emt)
    return out[0, 0], out[0, 1]


def _prep(inputs, em):
    xs = inputs * (LOG2E / BETA)
    tail = jnp.pad(em[CF:], ((0, CP - C), (0, 0)))
    col = jnp.arange(BC, dtype=jnp.int32)[None, :]
    bias = jnp.where(col < C - CF, 0.0, PADB).astype(jnp.float32)
    return xs, tail, bias


def kernel(inputs, targets, em, epoch):
    xs, tail, bias = _prep(inputs, em)
    emt = _gather_target_rows(em, targets)
    smooth, plain = _tc_losses(xs, em, tail, bias, emt)
    return jnp.where(epoch > 4, smooth, plain)


# speedup vs baseline: 1.0346x; 1.0013x over previous
import functools

import jax
import jax.numpy as jnp
from jax import lax
from jax.experimental import pallas as pl
from jax.experimental.pallas import tpu as pltpu
from jax.experimental.pallas import tpu_sc as plsc

C = 100000
F = 128
B = 1024
BETA = 0.05
K = 6
BC = 2048
NB = 49
CP = NB * BC
CF = (NB - 1) * BC
NCH = BC // 128
HF = F // 2
RT = 16
NEG = -3.0e38
PADB = -1.0e30
LOG2E = 1.4426950408889634
LN2 = 0.6931471805599453


def _gather_target_rows(em, targets):
    info = plsc.get_sparse_core_info()
    nw = info.num_cores * info.num_subcores
    bpw = B // nw
    mesh = plsc.VectorSubcoreMesh(core_axis_name="c", subcore_axis_name="s")

    @functools.partial(
        pl.kernel,
        mesh=mesh,
        out_type=jax.ShapeDtypeStruct((B, F), jnp.float32),
        scratch_types=[
            pltpu.VMEM((bpw,), jnp.int32),
            pltpu.VMEM((bpw, F), jnp.float32),
            pltpu.SemaphoreType.DMA,
        ],
    )
    def gather_kernel(em_hbm, idx_hbm, out_hbm, idx_v, rows_v, sem):
        wid = lax.axis_index("s") * info.num_cores + lax.axis_index("c")
        base = wid * bpw
        pltpu.sync_copy(idx_hbm.at[pl.ds(base, bpw)], idx_v)
        pltpu.async_copy(em_hbm.at[idx_v], rows_v, sem).wait()
        pltpu.sync_copy(rows_v, out_hbm.at[pl.ds(base, bpw)])

    return gather_kernel(em, targets)


def _body(xs_ref, em_ref, tail_ref, bias_ref, emt_ref, out_ref,
          m_ref, acc_ref, kb0, kb1, tau0, tau1, ks0, ks1, ps0, ps1):
    p = pl.program_id(0)
    j = pl.program_id(1)
    xs = xs_ref[...]
    x0 = xs[:, :HF]
    x1 = xs[:, HF:]
    dn = (((1,), (1,)), ((), ()))

    def sweep0(s0, s1):
        sim = s0 + s1
        bmax = jnp.max(sim, axis=1, keepdims=True)
        m_old = m_ref[...]
        m_new = jnp.maximum(m_old, bmax)
        ex = jnp.exp2(sim - m_new)
        ones = jnp.ones((BC,), jnp.float32)
        dnv = (((1,), (0,)), ((), ()))
        exs = lax.dot_general(ex, ones, dnv,
                              preferred_element_type=jnp.float32)
        acc_ref[...] = (acc_ref[...] * jnp.exp2(m_old - m_new)
                        + exs[:, None])
        m_ref[...] = m_new
        for rt in range(B // RT):
            lo = rt * RT
            hi_r = lo + RT
            for s, kb in ((s0, kb0), (s1, kb1)):
                r = [kb[i, lo:hi_r, :] for i in range(K)]
                for c in range(0, NCH, 2):
                    ea = s[lo:hi_r, c * 128:(c + 1) * 128]
                    eb = s[lo:hi_r, (c + 1) * 128:(c + 2) * 128]
                    h = jnp.maximum(ea, eb)
                    l = jnp.minimum(ea, eb)
                    e0 = jnp.maximum(r[0], h)
                    m = jnp.minimum(r[0], h)
                    e1 = jnp.maximum(r[2], m)
                    m = jnp.minimum(r[2], m)
                    e2 = jnp.maximum(r[4], m)
                    e3 = jnp.minimum(r[4], m)
                    o0 = jnp.maximum(r[1], l)
                    n = jnp.minimum(r[1], l)
                    o1 = jnp.maximum(r[3], n)
                    n = jnp.minimum(r[3], n)
                    o2 = jnp.maximum(r[5], n)
                    z1 = jnp.maximum(o0, e1)
                    z2 = jnp.minimum(o0, e1)
                    z3 = jnp.maximum(o1, e2)
                    z4 = jnp.minimum(o1, e2)
                    z5 = jnp.maximum(o2, e3)
                    r = [e0, z1, z2, z3, z4, z5]
                for i in range(K):
                    kb[i, lo:hi_r, :] = r[i]

    def sweep1(s0, s1):
        t0 = tau0[...]
        t1 = tau1[...]
        ones = jnp.ones((BC,), jnp.float32)
        dnv = (((1,), (0,)), ((), ()))
        q0 = lax.dot_general(jnp.where(s0 >= t0, s1, 0.0), ones, dnv,
                             preferred_element_type=jnp.float32)
        q1 = lax.dot_general(jnp.where(s1 >= t1, s0, 0.0), ones, dnv,
                             preferred_element_type=jnp.float32)
        ps0[...] = ps0[...] + q0[:, None]
        ps1[...] = ps1[...] + q1[:, None]

    @pl.when(j < NB - 1)
    def _clean():
        emb = em_ref[...]
        s0 = lax.dot_general(x0, emb[:, :HF], dn,
                             preferred_element_type=jnp.float32)
        s1 = lax.dot_general(x1, emb[:, HF:], dn,
                             preferred_element_type=jnp.float32)

        @pl.when(jnp.logical_and(p == 0, j == 0))
        def _init():
            m_ref[...] = jnp.full((B, 1), NEG, jnp.float32)
            acc_ref[...] = jnp.zeros((B, 1), jnp.float32)
            kb0[...] = jnp.full((K, B, 128), NEG, jnp.float32)
            kb1[...] = jnp.full((K, B, 128), NEG, jnp.float32)
            ps0[...] = jnp.zeros((B, 1), jnp.float32)
            ps1[...] = jnp.zeros((B, 1), jnp.float32)

        @pl.when(p == 0)
        def _p0():
            sweep0(s0, s1)

        @pl.when(p == 1)
        def _p1():
            @pl.when(j == 0)
            def _finalize_tau():
                for kb, tau, ks in ((kb0, tau0, ks0), (kb1, tau1, ks1)):
                    cand = jnp.concatenate(
                        [kb[i, :, :] for i in range(K)], axis=1)
                    ksum = jnp.zeros((B, 1), jnp.float32)
                    for t in range(K):
                        mt = jnp.max(cand, axis=1, keepdims=True)
                        ksum = ksum + mt
                        if t < K - 1:
                            cand = jnp.where(cand == mt, NEG, cand)
                        else:
                            tau[...] = mt
                    ks[...] = ksum

            sweep1(s0, s1)

    @pl.when(j == NB - 1)
    def _tail():
        emb = tail_ref[...]
        bias = bias_ref[...]
        s0 = lax.dot_general(x0, emb[:, :HF], dn,
                             preferred_element_type=jnp.float32) + bias
        s1 = lax.dot_general(x1, emb[:, HF:], dn,
                             preferred_element_type=jnp.float32) + bias

        @pl.when(p == 0)
        def _p0():
            sweep0(s0, s1)

        @pl.when(p == 1)
        def _p1():
            sweep1(s0, s1)
            lse = m_ref[...] + jnp.log(acc_ref[...]) * LOG2E
            prod = xs * emt_ref[...]
            st0 = jnp.sum(prod[:, :HF], axis=1, keepdims=True)
            st1 = jnp.sum(prod[:, HF:], axis=1, keepdims=True)
            sim_t = st0 + st1
            logp_t = sim_t - lse
            sum_logp0 = ks0[...] + ps0[...] - K * lse
            sum_logp1 = ks1[...] + ps1[...] - K * lse
            in0 = (st0 >= tau0[...]).astype(jnp.float32)
            in1 = (st1 >= tau1[...]).astype(jnp.float32)
            inv_k = 1.0 / K
            l0 = -logp_t - inv_k * (sum_logp0 - logp_t * in0)
            l1 = -logp_t - inv_k * (sum_logp1 - logp_t * in1)
            smooth = jnp.sum(0.5 * (l0 + l1)) * (LN2 / B)
            plain = jnp.sum(-logp_t) * (LN2 / B)
            lane = lax.broadcasted_iota(jnp.int32, (1, 128), 1)
            out_ref[...] = jnp.where(
                lane == 0, smooth, jnp.where(lane == 1, plain, 0.0))


def _tc_losses(xs, em, tail, bias, emt, interpret=False):
    out = pl.pallas_call(
        _body,
        grid=(2, NB),
        in_specs=[
            pl.BlockSpec((B, F), lambda p, j: (0, 0)),
            pl.BlockSpec((BC, F), lambda p, j: (jnp.minimum(j, NB - 2), 0)),
            pl.BlockSpec((BC, F), lambda p, j: (0, 0)),
            pl.BlockSpec((1, BC), lambda p, j: (0, 0)),
            pl.BlockSpec((B, F), lambda p, j: (0, 0)),
        ],
        out_specs=pl.BlockSpec((1, 128), lambda p, j: (0, 0)),
        out_shape=jax.ShapeDtypeStruct((1, 128), jnp.float32),
        scratch_shapes=[
            pltpu.VMEM((B, 1), jnp.float32),
            pltpu.VMEM((B, 1), jnp.float32),
            pltpu.VMEM((K, B, 128), jnp.float32),
            pltpu.VMEM((K, B, 128), jnp.float32),
            pltpu.VMEM((B, 1), jnp.float32),
            pltpu.VMEM((B, 1), jnp.float32),
            pltpu.VMEM((B, 1), jnp.float32),
            pltpu.VMEM((B, 1), jnp.float32),
            pltpu.VMEM((B, 1), jnp.float32),
            pltpu.VMEM((B, 1), jnp.float32),
        ],
        compiler_params=pltpu.CompilerParams(
            dimension_semantics=("arbitrary", "arbitrary")),
        interpret=interpret,
    )(xs, em, tail, bias, emt)
    return out[0, 0], out[0, 1]


def _prep(inputs, em):
    xs = inputs * (LOG2E / BETA)
    tail = jnp.pad(em[CF:], ((0, CP - C), (0, 0)))
    col = jnp.arange(BC, dtype=jnp.int32)[None, :]
    bias = jnp.where(col < C - CF, 0.0, PADB).astype(jnp.float32)
    return xs, tail, bias


def kernel(inputs, targets, em, epoch):
    xs, tail, bias = _prep(inputs, em)
    emt = _gather_target_rows(em, targets)
    smooth, plain = _tc_losses(xs, em, tail, bias, emt)
    return jnp.where(epoch > 4, smooth, plain)


# speedup vs baseline: 1.0348x; 1.0002x over previous
import functools

import jax
import jax.numpy as jnp
from jax import lax
from jax.experimental import pallas as pl
from jax.experimental.pallas import tpu as pltpu
from jax.experimental.pallas import tpu_sc as plsc

C = 100000
F = 128
B = 1024
BETA = 0.05
K = 6
BC = 2048
NB = 49
CP = NB * BC
CF = (NB - 1) * BC
NCH = BC // 128
HF = F // 2
RT = 8
NEG = -3.0e38
PADB = -1.0e30
LOG2E = 1.4426950408889634
LN2 = 0.6931471805599453


def _gather_target_rows(em, targets):
    info = plsc.get_sparse_core_info()
    nw = info.num_cores * info.num_subcores
    bpw = B // nw
    mesh = plsc.VectorSubcoreMesh(core_axis_name="c", subcore_axis_name="s")

    @functools.partial(
        pl.kernel,
        mesh=mesh,
        out_type=jax.ShapeDtypeStruct((B, F), jnp.float32),
        scratch_types=[
            pltpu.VMEM((bpw,), jnp.int32),
            pltpu.VMEM((bpw, F), jnp.float32),
            pltpu.SemaphoreType.DMA,
        ],
    )
    def gather_kernel(em_hbm, idx_hbm, out_hbm, idx_v, rows_v, sem):
        wid = lax.axis_index("s") * info.num_cores + lax.axis_index("c")
        base = wid * bpw
        pltpu.sync_copy(idx_hbm.at[pl.ds(base, bpw)], idx_v)
        pltpu.async_copy(em_hbm.at[idx_v], rows_v, sem).wait()
        pltpu.sync_copy(rows_v, out_hbm.at[pl.ds(base, bpw)])

    return gather_kernel(em, targets)


def _body(xs_ref, em_ref, tail_ref, bias_ref, emt_ref, out_ref,
          m_ref, acc_ref, kb0, kb1, tau0, tau1, ks0, ks1, ps0, ps1):
    p = pl.program_id(0)
    j = pl.program_id(1)
    xs = xs_ref[...]
    x0 = xs[:, :HF]
    x1 = xs[:, HF:]
    dn = (((1,), (1,)), ((), ()))

    def sweep0(s0, s1):
        sim = s0 + s1
        bmax = jnp.max(sim, axis=1, keepdims=True)
        m_old = m_ref[...]
        m_new = jnp.maximum(m_old, bmax)
        ex = jnp.exp2(sim - m_new)
        ones = jnp.ones((BC,), jnp.float32)
        dnv = (((1,), (0,)), ((), ()))
        exs = lax.dot_general(ex, ones, dnv,
                              preferred_element_type=jnp.float32)
        acc_ref[...] = (acc_ref[...] * jnp.exp2(m_old - m_new)
                        + exs[:, None])
        m_ref[...] = m_new
        for rt in range(B // RT):
            lo = rt * RT
            hi_r = lo + RT
            for s, kb in ((s0, kb0), (s1, kb1)):
                r = [kb[i, lo:hi_r, :] for i in range(K)]
                for c in range(0, NCH, 2):
                    ea = s[lo:hi_r, c * 128:(c + 1) * 128]
                    eb = s[lo:hi_r, (c + 1) * 128:(c + 2) * 128]
                    h = jnp.maximum(ea, eb)
                    l = jnp.minimum(ea, eb)
                    e0 = jnp.maximum(r[0], h)
                    m = jnp.minimum(r[0], h)
                    e1 = jnp.maximum(r[2], m)
                    m = jnp.minimum(r[2], m)
                    e2 = jnp.maximum(r[4], m)
                    e3 = jnp.minimum(r[4], m)
                    o0 = jnp.maximum(r[1], l)
                    n = jnp.minimum(r[1], l)
                    o1 = jnp.maximum(r[3], n)
                    n = jnp.minimum(r[3], n)
                    o2 = jnp.maximum(r[5], n)
                    z1 = jnp.maximum(o0, e1)
                    z2 = jnp.minimum(o0, e1)
                    z3 = jnp.maximum(o1, e2)
                    z4 = jnp.minimum(o1, e2)
                    z5 = jnp.maximum(o2, e3)
                    r = [e0, z1, z2, z3, z4, z5]
                for i in range(K):
                    kb[i, lo:hi_r, :] = r[i]

    def sweep1(s0, s1):
        t0 = tau0[...]
        t1 = tau1[...]
        ones = jnp.ones((BC,), jnp.float32)
        dnv = (((1,), (0,)), ((), ()))
        q0 = lax.dot_general(jnp.where(s0 >= t0, s1, 0.0), ones, dnv,
                             preferred_element_type=jnp.float32)
        q1 = lax.dot_general(jnp.where(s1 >= t1, s0, 0.0), ones, dnv,
                             preferred_element_type=jnp.float32)
        ps0[...] = ps0[...] + q0[:, None]
        ps1[...] = ps1[...] + q1[:, None]

    @pl.when(j < NB - 1)
    def _clean():
        emb = em_ref[...]
        s0 = lax.dot_general(x0, emb[:, :HF], dn,
                             preferred_element_type=jnp.float32)
        s1 = lax.dot_general(x1, emb[:, HF:], dn,
                             preferred_element_type=jnp.float32)

        @pl.when(jnp.logical_and(p == 0, j == 0))
        def _init():
            m_ref[...] = jnp.full((B, 1), NEG, jnp.float32)
            acc_ref[...] = jnp.zeros((B, 1), jnp.float32)
            kb0[...] = jnp.full((K, B, 128), NEG, jnp.float32)
            kb1[...] = jnp.full((K, B, 128), NEG, jnp.float32)
            ps0[...] = jnp.zeros((B, 1), jnp.float32)
            ps1[...] = jnp.zeros((B, 1), jnp.float32)

        @pl.when(p == 0)
        def _p0():
            sweep0(s0, s1)

        @pl.when(p == 1)
        def _p1():
            @pl.when(j == 0)
            def _finalize_tau():
                for kb, tau, ks in ((kb0, tau0, ks0), (kb1, tau1, ks1)):
                    cand = jnp.concatenate(
                        [kb[i, :, :] for i in range(K)], axis=1)
                    ksum = jnp.zeros((B, 1), jnp.float32)
                    for t in range(K):
                        mt = jnp.max(cand, axis=1, keepdims=True)
                        ksum = ksum + mt
                        if t < K - 1:
                            cand = jnp.where(cand == mt, NEG, cand)
                        else:
                            tau[...] = mt
                    ks[...] = ksum

            sweep1(s0, s1)

    @pl.when(j == NB - 1)
    def _tail():
        emb = tail_ref[...]
        bias = bias_ref[...]
        s0 = lax.dot_general(x0, emb[:, :HF], dn,
                             preferred_element_type=jnp.float32) + bias
        s1 = lax.dot_general(x1, emb[:, HF:], dn,
                             preferred_element_type=jnp.float32) + bias

        @pl.when(p == 0)
        def _p0():
            sweep0(s0, s1)

        @pl.when(p == 1)
        def _p1():
            sweep1(s0, s1)
            lse = m_ref[...] + jnp.log(acc_ref[...]) * LOG2E
            prod = xs * emt_ref[...]
            st0 = jnp.sum(prod[:, :HF], axis=1, keepdims=True)
            st1 = jnp.sum(prod[:, HF:], axis=1, keepdims=True)
            sim_t = st0 + st1
            logp_t = sim_t - lse
            sum_logp0 = ks0[...] + ps0[...] - K * lse
            sum_logp1 = ks1[...] + ps1[...] - K * lse
            in0 = (st0 >= tau0[...]).astype(jnp.float32)
            in1 = (st1 >= tau1[...]).astype(jnp.float32)
            inv_k = 1.0 / K
            l0 = -logp_t - inv_k * (sum_logp0 - logp_t * in0)
            l1 = -logp_t - inv_k * (sum_logp1 - logp_t * in1)
            smooth = jnp.sum(0.5 * (l0 + l1)) * (LN2 / B)
            plain = jnp.sum(-logp_t) * (LN2 / B)
            lane = lax.broadcasted_iota(jnp.int32, (1, 128), 1)
            out_ref[...] = jnp.where(
                lane == 0, smooth, jnp.where(lane == 1, plain, 0.0))


def _tc_losses(xs, em, tail, bias, emt, interpret=False):
    out = pl.pallas_call(
        _body,
        grid=(2, NB),
        in_specs=[
            pl.BlockSpec((B, F), lambda p, j: (0, 0)),
            pl.BlockSpec((BC, F), lambda p, j: (jnp.minimum(j, NB - 2), 0)),
            pl.BlockSpec((BC, F), lambda p, j: (0, 0)),
            pl.BlockSpec((1, BC), lambda p, j: (0, 0)),
            pl.BlockSpec((B, F), lambda p, j: (0, 0)),
        ],
        out_specs=pl.BlockSpec((1, 128), lambda p, j: (0, 0)),
        out_shape=jax.ShapeDtypeStruct((1, 128), jnp.float32),
        scratch_shapes=[
            pltpu.VMEM((B, 1), jnp.float32),
            pltpu.VMEM((B, 1), jnp.float32),
            pltpu.VMEM((K, B, 128), jnp.float32),
            pltpu.VMEM((K, B, 128), jnp.float32),
            pltpu.VMEM((B, 1), jnp.float32),
            pltpu.VMEM((B, 1), jnp.float32),
            pltpu.VMEM((B, 1), jnp.float32),
            pltpu.VMEM((B, 1), jnp.float32),
            pltpu.VMEM((B, 1), jnp.float32),
            pltpu.VMEM((B, 1), jnp.float32),
        ],
        compiler_params=pltpu.CompilerParams(
            dimension_semantics=("arbitrary", "arbitrary")),
        interpret=interpret,
    )(xs, em, tail, bias, emt)
    return out[0, 0], out[0, 1]


def _prep(inputs, em):
    xs = inputs * (LOG2E / BETA)
    tail = jnp.pad(em[CF:], ((0, CP - C), (0, 0)))
    col = jnp.arange(BC, dtype=jnp.int32)[None, :]
    bias = jnp.where(col < C - CF, 0.0, PADB).astype(jnp.float32)
    return xs, tail, bias


def kernel(inputs, targets, em, epoch):
    xs, tail, bias = _prep(inputs, em)
    emt = _gather_target_rows(em, targets)
    smooth, plain = _tc_losses(xs, em, tail, bias, emt)
    return jnp.where(epoch > 4, smooth, plain)
